# baseline (speedup 1.0000x reference)
import functools
import math

import jax
import jax.numpy as jnp
from jax import lax
from jax.experimental import pallas as pl
from jax.experimental.pallas import tpu as pltpu
from jax.experimental.pallas import tpu_sc as plsc

E = 8
L2 = 512
L3 = 32
DR = 2048
DE = 2048
B = 4096

BT = 512
NBT = B // BT

TB = 256
T_MAX = B // TB + E - 1
TRASH_BLK = (B + E * TB) // TB
XS_ROWS = (TRASH_BLK + 1) * TB

NW = 32
CHUNK = B // NW
ROWB = 16
NVEC = B // 16


def _router_body(x_ref, rwt_ref, rb_ref,
                 idx_ref, frac_ref, avg_ref, aux_ref, z_ref, ent_ref, top_ref):
    i = pl.program_id(0)
    x = x_ref[...]
    logits = jnp.dot(x, rwt_ref[...], preferred_element_type=jnp.float32)
    logits = logits + rb_ref[...]
    m = jnp.max(logits, axis=-1, keepdims=True)
    ex = jnp.exp(logits - m)
    s = jnp.sum(ex, axis=-1, keepdims=True)
    p = ex / s
    lse = m + jnp.log(s)

    iota = lax.broadcasted_iota(jnp.int32, (BT, E), 1)
    idx = jnp.min(jnp.where(logits == m, iota, E), axis=-1, keepdims=True)
    idx_ref[...] = idx

    onehot = (iota == idx).astype(jnp.float32)
    part_p = jnp.sum(p, axis=0, keepdims=True)
    part_c = jnp.sum(onehot, axis=0, keepdims=True)
    part_z = jnp.sum(lse * lse)[None, None]
    part_e = jnp.sum(-p * jnp.log(jnp.maximum(p, 1e-9)))[None, None]
    part_t = jnp.sum(jnp.max(p, axis=-1))[None, None]

    @pl.when(i == 0)
    def _init():
        avg_ref[...] = part_p
        frac_ref[...] = part_c
        z_ref[...] = part_z
        ent_ref[...] = part_e
        top_ref[...] = part_t

    @pl.when(i != 0)
    def _acc():
        avg_ref[...] += part_p
        frac_ref[...] += part_c
        z_ref[...] += part_z
        ent_ref[...] += part_e
        top_ref[...] += part_t

    @pl.when(i == NBT - 1)
    def _fin():
        frac = frac_ref[...] / float(B)
        avg = avg_ref[...] / float(B)
        frac_ref[...] = frac
        avg_ref[...] = avg
        aux_ref[...] = (float(E) * jnp.sum(frac * avg))[None, None]
        z_ref[...] = z_ref[...] / float(B)
        ent_ref[...] = ent_ref[...] / (float(B) * math.log(float(E)))
        top_ref[...] = top_ref[...] / float(B)


def _router(router_input, rwt, rb):
    f32 = jnp.float32
    out_shapes = (
        jax.ShapeDtypeStruct((B, 1), jnp.int32),
        jax.ShapeDtypeStruct((1, E), f32),
        jax.ShapeDtypeStruct((1, E), f32),
        jax.ShapeDtypeStruct((1, 1), f32),
        jax.ShapeDtypeStruct((1, 1), f32),
        jax.ShapeDtypeStruct((1, 1), f32),
        jax.ShapeDtypeStruct((1, 1), f32),
    )
    const = lambda i: (0, 0)
    return pl.pallas_call(
        _router_body,
        grid=(NBT,),
        in_specs=[
            pl.BlockSpec((BT, DR), lambda i: (i, 0)),
            pl.BlockSpec((DR, E), const),
            pl.BlockSpec((1, E), const),
        ],
        out_specs=(
            pl.BlockSpec((BT, 1), lambda i: (i, 0)),
            pl.BlockSpec((1, E), const),
            pl.BlockSpec((1, E), const),
            pl.BlockSpec((1, 1), const),
            pl.BlockSpec((1, 1), const),
            pl.BlockSpec((1, 1), const),
            pl.BlockSpec((1, 1), const),
        ),
        out_shape=out_shapes,
    )(router_input, rwt, rb)


def _dispatch_body(idx_hbm, x_hbm,
                   xs_hbm, dst_hbm, te_hbm, tb_hbm,
                   idx_all, cnt_ref, dstc_ref, dstl_ref, rows_ref,
                   me_ref, mb_ref, sem):
    c = lax.axis_index("c")
    s = lax.axis_index("s")
    w = s * 2 + c
    base_tok = w * CHUNK

    pltpu.sync_copy(idx_hbm, idx_all)

    lanes = lax.iota(jnp.int32, 16)

    def hist_step(j, counts):
        v = idx_all[pl.ds(j * 16, 16)]
        for e in range(E):
            pop = jnp.sum((v == e).astype(jnp.int32))
            counts = counts + jnp.where(lanes == e, pop, 0)
        return counts

    zero16 = jnp.zeros((16,), jnp.int32)
    prefix = lax.fori_loop(0, w * (CHUNK // 16), hist_step, zero16)
    total = lax.fori_loop(w * (CHUNK // 16), NVEC, hist_step, prefix)

    pc = ((total + (TB - 1)) >> 8) << 8
    seg_incl = plsc.cumsum(pc)
    seg_start = seg_incl - pc
    cnt_ref[...] = seg_start + prefix

    for j in range(CHUNK // 16):
        v = idx_all[pl.ds((w * (CHUNK // 16) + j) * 16, 16)]
        pos = plsc.load_gather(cnt_ref, [v])
        riv = zero16
        newcnt = cnt_ref[...]
        for e in range(E):
            mask = v == e
            cs = plsc.cumsum(mask.astype(jnp.int32))
            riv = jnp.where(mask, cs - 1, riv)
            newcnt = newcnt + jnp.where(lanes == e, cs[15], 0)
        cnt_ref[...] = newcnt
        dst_v = pos + riv
        dstc_ref[j] = dst_v
        dstl_ref[pl.ds(j * 16, 16)] = dst_v
    pltpu.sync_copy(dstl_ref, dst_hbm.at[pl.ds(base_tok, CHUNK)])

    for bi in range(CHUNK // ROWB):
        pltpu.sync_copy(x_hbm.at[pl.ds(base_tok + bi * ROWB, ROWB)], rows_ref)
        pltpu.async_copy(rows_ref, xs_hbm.at[dstc_ref.at[bi]], sem).wait()

    @pl.when(w == 0)
    def _meta():
        nt = pc >> 8
        tiles_before = plsc.cumsum(nt) - nt
        segblk = seg_start >> 8
        for half in range(2):
            t_iota = lanes + half * 16
            me = jnp.zeros((16,), jnp.int32)
            mb = jnp.full((16,), TRASH_BLK, jnp.int32)
            for e in range(E):
                tb_e = jnp.sum(jnp.where(lanes == e, tiles_before, 0))
                nt_e = jnp.sum(jnp.where(lanes == e, nt, 0))
                sb_e = jnp.sum(jnp.where(lanes == e, segblk, 0))
                m = (t_iota >= tb_e) & (t_iota < tb_e + nt_e)
                me = jnp.where(m, e, me)
                mb = jnp.where(m, sb_e + t_iota - tb_e, mb)
            me_ref[pl.ds(half * 16, 16)] = me
            mb_ref[pl.ds(half * 16, 16)] = mb
        pltpu.sync_copy(me_ref, te_hbm)
        pltpu.sync_copy(mb_ref, tb_hbm)


@functools.cache
def _dispatch():
    return pl.kernel(
        _dispatch_body,
        out_type=(
            jax.ShapeDtypeStruct((XS_ROWS, DE), jnp.float32),
            jax.ShapeDtypeStruct((B,), jnp.int32),
            jax.ShapeDtypeStruct((NW,), jnp.int32),
            jax.ShapeDtypeStruct((NW,), jnp.int32),
        ),
        mesh=plsc.VectorSubcoreMesh(core_axis_name="c", subcore_axis_name="s"),
        compiler_params=pltpu.CompilerParams(
            needs_layout_passes=False, use_tc_tiling_on_sc=True),
        scratch_types=[
            pltpu.VMEM((B,), jnp.int32),
            pltpu.VMEM((16,), jnp.int32),
            pltpu.VMEM((CHUNK // ROWB, ROWB), jnp.int32),
            pltpu.VMEM((CHUNK,), jnp.int32),
            pltpu.VMEM((ROWB, DE), jnp.float32),
            pltpu.VMEM((NW,), jnp.int32),
            pltpu.VMEM((NW,), jnp.int32),
            pltpu.SemaphoreType.DMA,
        ],
    )


def _grouped_body(te_ref, tb_ref, x_ref, w1_ref, b1_ref, w2_ref, b2_ref,
                  ow_ref, ob_ref, out_ref):
    x = x_ref[...]
    w1 = w1_ref[0]
    h = jnp.dot(x, w1, preferred_element_type=jnp.float32) + b1_ref[0]
    ha = h[:, :L2]
    hx = h[:, L2:L2 + 1]
    pa = jnp.clip(ha * ha * (255.0 / 256.0), 0.0, 1.0)
    qa = jnp.clip(ha, 0.0, 1.0)
    w2 = w2_ref[0]
    y = (jnp.dot(pa, w2[:L2], preferred_element_type=jnp.float32)
         + jnp.dot(qa, w2[L2:], preferred_element_type=jnp.float32)
         + b2_ref[0])
    y = jnp.clip(y, 0.0, 1.0)
    o = jnp.dot(y, ow_ref[0], preferred_element_type=jnp.float32)
    out_ref[...] = o + ob_ref[0] + hx


def _grouped(te, tb, xs, w1t, b1, w2t, b2, owt, ob):
    grid_spec = pltpu.PrefetchScalarGridSpec(
        num_scalar_prefetch=2,
        grid=(T_MAX,),
        in_specs=[
            pl.BlockSpec((TB, DE), lambda t, te, tb: (tb[t], 0)),
            pl.BlockSpec((1, DE, L2 + 1), lambda t, te, tb: (te[t], 0, 0)),
            pl.BlockSpec((1, 1, L2 + 1), lambda t, te, tb: (te[t], 0, 0)),
            pl.BlockSpec((1, 2 * L2, L3), lambda t, te, tb: (te[t], 0, 0)),
            pl.BlockSpec((1, 1, L3), lambda t, te, tb: (te[t], 0, 0)),
            pl.BlockSpec((1, L3, 1), lambda t, te, tb: (te[t], 0, 0)),
            pl.BlockSpec((1, 1, 1), lambda t, te, tb: (te[t], 0, 0)),
        ],
        out_specs=pl.BlockSpec((TB, 1), lambda t, te, tb: (tb[t], 0)),
    )
    return pl.pallas_call(
        _grouped_body,
        grid_spec=grid_spec,
        out_shape=jax.ShapeDtypeStruct((XS_ROWS, 1), jnp.float32),
    )(te, tb, xs, w1t, b1, w2t, b2, owt, ob)


def _unpermute_body(os_hbm, dst_hbm, out_hbm, os_v, dst_v, res_v):
    c = lax.axis_index("c")
    s = lax.axis_index("s")
    w = s * 2 + c
    base = w * CHUNK
    pltpu.sync_copy(os_hbm, os_v)
    pltpu.sync_copy(dst_hbm.at[pl.ds(base, CHUNK)], dst_v)
    for j in range(CHUNK // 16):
        iv = dst_v[pl.ds(j * 16, 16)]
        res_v[pl.ds(j * 16, 16)] = plsc.load_gather(os_v, [iv])
    pltpu.sync_copy(res_v, out_hbm.at[pl.ds(base, CHUNK)])


@functools.cache
def _unpermute():
    return pl.kernel(
        _unpermute_body,
        out_type=jax.ShapeDtypeStruct((B,), jnp.float32),
        mesh=plsc.VectorSubcoreMesh(core_axis_name="c", subcore_axis_name="s"),
        compiler_params=pltpu.CompilerParams(needs_layout_passes=False),
        scratch_types=[
            pltpu.VMEM((XS_ROWS,), jnp.float32),
            pltpu.VMEM((CHUNK,), jnp.int32),
            pltpu.VMEM((CHUNK,), jnp.float32),
        ],
    )


@jax.jit
def kernel(expert_input, router_input, router_w, router_b,
           l1_w, l1_b, l2_w, l2_b, out_w, out_b):
    f32 = jnp.float32
    rwt = jnp.swapaxes(router_w.astype(f32), 0, 1)
    rb = router_b.astype(f32).reshape(1, E)
    idx, frac, avg, aux, z, ent, top = _router(router_input.astype(f32), rwt, rb)

    xs, dst, te, tb = _dispatch()(idx.reshape(B), expert_input)

    w1t = jnp.swapaxes(l1_w, 1, 2)
    w2t = jnp.swapaxes(l2_w, 1, 2)
    owt = jnp.swapaxes(out_w, 1, 2)
    os = _grouped(te, tb, xs, w1t, l1_b.reshape(E, 1, L2 + 1), w2t,
                  l2_b.reshape(E, 1, L3), owt, out_b.reshape(E, 1, 1))

    l3x = _unpermute()(os.reshape(XS_ROWS), dst).reshape(B, 1)

    zero = jnp.zeros((), dtype=expert_input.dtype)
    return (l3x, zero, aux.reshape(()), z.reshape(()), frac.reshape(E),
            avg.reshape(E), ent.reshape(()), top.reshape(()), zero)


# speedup vs baseline: 1.2205x; 1.2205x over previous
import functools
import math

import jax
import jax.numpy as jnp
from jax import lax
from jax.experimental import pallas as pl
from jax.experimental.pallas import tpu as pltpu
from jax.experimental.pallas import tpu_sc as plsc

E = 8
L2 = 512
L3 = 32
DR = 2048
DE = 2048
B = 4096

BT = 512
NBT = B // BT

TB = 512
TB_BITS = 9
T_MAX = B // TB + E - 1
TRASH_BLK = (B + E * TB) // TB
XS_ROWS = (TRASH_BLK + 1) * TB

NW = 32
CHUNK = B // NW
ROWB = 16
NVEC = B // 16


def _router_body(x_ref, rwt_ref, rb_ref,
                 idx_ref, frac_ref, avg_ref, aux_ref, z_ref, ent_ref, top_ref,
                 pref_ref, cnt_ref, run_ref):
    i = pl.program_id(0)
    x = x_ref[...]
    logits = lax.dot_general(x, rwt_ref[...], (((1,), (1,)), ((), ())),
                             preferred_element_type=jnp.float32)
    logits = logits + rb_ref[...]
    m = jnp.max(logits, axis=-1, keepdims=True)
    ex = jnp.exp(logits - m)
    s = jnp.sum(ex, axis=-1, keepdims=True)
    p = ex / s
    lse = m + jnp.log(s)

    iota = lax.broadcasted_iota(jnp.int32, (BT, E), 1)
    idx = jnp.min(jnp.where(logits == m, iota, E), axis=-1, keepdims=True)
    idx_ref[...] = idx

    iota16 = lax.broadcasted_iota(jnp.int32, (BT, 16), 1)
    oh16 = (iota16 == idx).astype(jnp.int32)
    subs = [jnp.sum(oh16[k * CHUNK:(k + 1) * CHUNK], axis=0, keepdims=True)
            for k in range(BT // CHUNK)]

    @pl.when(i == 0)
    def _init_run():
        run_ref[...] = jnp.zeros((1, 16), jnp.int32)

    rows = [run_ref[...]]
    for k in range(BT // CHUNK - 1):
        rows.append(rows[-1] + subs[k])
    pref_ref[0] = jnp.concatenate(rows, axis=0)
    run_ref[...] = rows[-1] + subs[BT // CHUNK - 1]

    @pl.when(i == NBT - 1)
    def _fin_cnt():
        cnt_ref[...] = run_ref[...]

    onehot = (iota == idx).astype(jnp.float32)
    part_p = jnp.sum(p, axis=0, keepdims=True)
    part_c = jnp.sum(onehot, axis=0, keepdims=True)
    part_z = jnp.sum(lse * lse)[None, None]
    part_e = jnp.sum(-p * jnp.log(jnp.maximum(p, 1e-9)))[None, None]
    part_t = jnp.sum(jnp.max(p, axis=-1))[None, None]

    @pl.when(i == 0)
    def _init():
        avg_ref[...] = part_p
        frac_ref[...] = part_c
        z_ref[...] = part_z
        ent_ref[...] = part_e
        top_ref[...] = part_t

    @pl.when(i != 0)
    def _acc():
        avg_ref[...] += part_p
        frac_ref[...] += part_c
        z_ref[...] += part_z
        ent_ref[...] += part_e
        top_ref[...] += part_t

    @pl.when(i == NBT - 1)
    def _fin():
        frac = frac_ref[...] / float(B)
        avg = avg_ref[...] / float(B)
        frac_ref[...] = frac
        avg_ref[...] = avg
        aux_ref[...] = (float(E) * jnp.sum(frac * avg))[None, None]
        z_ref[...] = z_ref[...] / float(B)
        ent_ref[...] = ent_ref[...] / (float(B) * math.log(float(E)))
        top_ref[...] = top_ref[...] / float(B)


def _router(router_input, rwt, rb):
    f32 = jnp.float32
    out_shapes = (
        jax.ShapeDtypeStruct((B, 1), jnp.int32),
        jax.ShapeDtypeStruct((1, E), f32),
        jax.ShapeDtypeStruct((1, E), f32),
        jax.ShapeDtypeStruct((1, 1), f32),
        jax.ShapeDtypeStruct((1, 1), f32),
        jax.ShapeDtypeStruct((1, 1), f32),
        jax.ShapeDtypeStruct((1, 1), f32),
        jax.ShapeDtypeStruct((NBT, BT // CHUNK, 16), jnp.int32),
        jax.ShapeDtypeStruct((1, 16), jnp.int32),
    )
    const = lambda i: (0, 0)
    return pl.pallas_call(
        _router_body,
        grid=(NBT,),
        in_specs=[
            pl.BlockSpec((BT, DR), lambda i: (i, 0)),
            pl.BlockSpec((E, DR), const),
            pl.BlockSpec((1, E), const),
        ],
        out_specs=(
            pl.BlockSpec((BT, 1), lambda i: (i, 0)),
            pl.BlockSpec((1, E), const),
            pl.BlockSpec((1, E), const),
            pl.BlockSpec((1, 1), const),
            pl.BlockSpec((1, 1), const),
            pl.BlockSpec((1, 1), const),
            pl.BlockSpec((1, 1), const),
            pl.BlockSpec((1, BT // CHUNK, 16), lambda i: (i, 0, 0)),
            pl.BlockSpec((1, 16), const),
        ),
        out_shape=out_shapes,
        scratch_shapes=[pltpu.VMEM((1, 16), jnp.int32)],
    )(router_input, rwt, rb)


def _dispatch_body(idx_hbm, x_hbm, pref_hbm, tot_hbm,
                   xs_hbm, dst_hbm, te_hbm, tb_hbm,
                   idx_all, cnt_ref, pfx_ref, tot_ref, dstc_ref, dstl_ref,
                   rows0_ref, rows1_ref, me_ref, mb_ref, sem_in, sem_out):
    c = lax.axis_index("c")
    s = lax.axis_index("s")
    w = s * 2 + c
    base_tok = w * CHUNK

    pltpu.sync_copy(idx_hbm.at[pl.ds(base_tok, CHUNK)], idx_all)
    pltpu.sync_copy(pref_hbm.at[w // 4, w % 4], pfx_ref)
    pltpu.sync_copy(tot_hbm.at[0], tot_ref)

    h_in0 = pltpu.make_async_copy(
        x_hbm.at[pl.ds(base_tok, ROWB)], rows0_ref, sem_in)
    h_in0.start()

    lanes = lax.iota(jnp.int32, 16)
    zero16 = jnp.zeros((16,), jnp.int32)
    total = tot_ref[...]
    prefix = pfx_ref[...]

    pc = ((total + (TB - 1)) >> TB_BITS) << TB_BITS
    seg_incl = plsc.cumsum(pc)
    seg_start = seg_incl - pc
    cnt_ref[...] = seg_start + prefix

    for j in range(CHUNK // 16):
        v = idx_all[pl.ds(j * 16, 16)]
        pos = plsc.load_gather(cnt_ref, [v])
        riv = zero16
        newcnt = cnt_ref[...]
        for e in range(E):
            mask = v == e
            cs = plsc.cumsum(mask.astype(jnp.int32))
            riv = jnp.where(mask, cs - 1, riv)
            newcnt = newcnt + jnp.where(lanes == e, cs[15], 0)
        cnt_ref[...] = newcnt
        dst_v = pos + riv
        dstc_ref[j] = dst_v
        dstl_ref[pl.ds(j * 16, 16)] = dst_v
    pltpu.sync_copy(dstl_ref, dst_hbm.at[pl.ds(base_tok, CHUNK)])

    bufs = (rows0_ref, rows1_ref)
    nb = CHUNK // ROWB
    h_out = [None] * nb
    h_in = [h_in0] + [None] * (nb - 1)
    for bi in range(nb):
        h_in[bi].wait()
        h_out[bi] = pltpu.make_async_copy(
            bufs[bi % 2], xs_hbm.at[dstc_ref.at[bi]], sem_out)
        h_out[bi].start()
        if bi + 1 < nb:
            if bi >= 1:
                h_out[bi - 1].wait()
            h_in[bi + 1] = pltpu.make_async_copy(
                x_hbm.at[pl.ds(base_tok + (bi + 1) * ROWB, ROWB)],
                bufs[(bi + 1) % 2], sem_in)
            h_in[bi + 1].start()
    h_out[nb - 2].wait()
    h_out[nb - 1].wait()

    @pl.when(w == 0)
    def _meta():
        nt = pc >> TB_BITS
        tiles_before = plsc.cumsum(nt) - nt
        segblk = seg_start >> TB_BITS
        for half in range(2):
            t_iota = lanes + half * 16
            me = jnp.zeros((16,), jnp.int32)
            mb = jnp.full((16,), TRASH_BLK, jnp.int32)
            for e in range(E):
                tb_e = jnp.sum(jnp.where(lanes == e, tiles_before, 0))
                nt_e = jnp.sum(jnp.where(lanes == e, nt, 0))
                sb_e = jnp.sum(jnp.where(lanes == e, segblk, 0))
                m = (t_iota >= tb_e) & (t_iota < tb_e + nt_e)
                me = jnp.where(m, e, me)
                mb = jnp.where(m, sb_e + t_iota - tb_e, mb)
            me_ref[pl.ds(half * 16, 16)] = me
            mb_ref[pl.ds(half * 16, 16)] = mb
        pltpu.sync_copy(me_ref, te_hbm)
        pltpu.sync_copy(mb_ref, tb_hbm)


@functools.cache
def _dispatch():
    return pl.kernel(
        _dispatch_body,
        out_type=(
            jax.ShapeDtypeStruct((XS_ROWS, DE), jnp.float32),
            jax.ShapeDtypeStruct((B,), jnp.int32),
            jax.ShapeDtypeStruct((NW,), jnp.int32),
            jax.ShapeDtypeStruct((NW,), jnp.int32),
        ),
        mesh=plsc.VectorSubcoreMesh(core_axis_name="c", subcore_axis_name="s"),
        compiler_params=pltpu.CompilerParams(
            needs_layout_passes=False, use_tc_tiling_on_sc=True),
        scratch_types=[
            pltpu.VMEM((CHUNK,), jnp.int32),
            pltpu.VMEM((16,), jnp.int32),
            pltpu.VMEM((16,), jnp.int32),
            pltpu.VMEM((16,), jnp.int32),
            pltpu.VMEM((CHUNK // ROWB, ROWB), jnp.int32),
            pltpu.VMEM((CHUNK,), jnp.int32),
            pltpu.VMEM((ROWB, DE), jnp.float32),
            pltpu.VMEM((ROWB, DE), jnp.float32),
            pltpu.VMEM((NW,), jnp.int32),
            pltpu.VMEM((NW,), jnp.int32),
            pltpu.SemaphoreType.DMA,
            pltpu.SemaphoreType.DMA,
        ],
    )


def _grouped_body(te_ref, tb_ref, x_ref, w1_ref, b1_ref, w2_ref, b2_ref,
                  ow_ref, ob_ref, out_ref):
    t = pl.program_id(0)

    @pl.when(tb_ref[t] != TRASH_BLK)
    def _work():
        nt = (((1,), (1,)), ((), ()))
        x = x_ref[...]
        w1 = w1_ref[0]
        ha = lax.dot_general(x, w1[:L2], nt, preferred_element_type=jnp.float32)
        ha = ha + b1_ref[0][:, :L2]
        hx = (jnp.sum(x * w1[L2:], axis=1, keepdims=True)
              + b1_ref[0][:, L2:])
        pa = jnp.clip(ha * ha * (255.0 / 256.0), 0.0, 1.0)
        qa = jnp.clip(ha, 0.0, 1.0)
        w2 = w2_ref[0]
        y = (lax.dot_general(pa, w2[:, :L2], nt,
                             preferred_element_type=jnp.float32)
             + lax.dot_general(qa, w2[:, L2:], nt,
                               preferred_element_type=jnp.float32)
             + b2_ref[0])
        y = jnp.clip(y, 0.0, 1.0)
        o = jnp.sum(y * ow_ref[0], axis=1, keepdims=True)
        out_ref[...] = o + ob_ref[0] + hx


def _grouped(te, tb, xs, w1t, b1, w2t, b2, owt, ob):
    grid_spec = pltpu.PrefetchScalarGridSpec(
        num_scalar_prefetch=2,
        grid=(T_MAX,),
        in_specs=[
            pl.BlockSpec((TB, DE), lambda t, te, tb: (tb[t], 0)),
            pl.BlockSpec((1, L2 + 1, DE), lambda t, te, tb: (te[t], 0, 0)),
            pl.BlockSpec((1, 1, L2 + 1), lambda t, te, tb: (te[t], 0, 0)),
            pl.BlockSpec((1, L3, 2 * L2), lambda t, te, tb: (te[t], 0, 0)),
            pl.BlockSpec((1, 1, L3), lambda t, te, tb: (te[t], 0, 0)),
            pl.BlockSpec((1, 1, L3), lambda t, te, tb: (te[t], 0, 0)),
            pl.BlockSpec((1, 1, 1), lambda t, te, tb: (te[t], 0, 0)),
        ],
        out_specs=pl.BlockSpec((TB, 1), lambda t, te, tb: (tb[t], 0)),
    )
    return pl.pallas_call(
        _grouped_body,
        grid_spec=grid_spec,
        out_shape=jax.ShapeDtypeStruct((XS_ROWS, 1), jnp.float32),
    )(te, tb, xs, w1t, b1, w2t, b2, owt, ob)


def _unpermute_body(os_hbm, dst_hbm, out_hbm, os_v, dst_v, res_v):
    c = lax.axis_index("c")
    s = lax.axis_index("s")
    w = s * 2 + c
    base = w * CHUNK
    pltpu.sync_copy(os_hbm, os_v)
    pltpu.sync_copy(dst_hbm.at[pl.ds(base, CHUNK)], dst_v)
    for j in range(CHUNK // 16):
        iv = dst_v[pl.ds(j * 16, 16)]
        res_v[pl.ds(j * 16, 16)] = plsc.load_gather(os_v, [iv])
    pltpu.sync_copy(res_v, out_hbm.at[pl.ds(base, CHUNK)])


@functools.cache
def _unpermute():
    return pl.kernel(
        _unpermute_body,
        out_type=jax.ShapeDtypeStruct((B,), jnp.float32),
        mesh=plsc.VectorSubcoreMesh(core_axis_name="c", subcore_axis_name="s"),
        compiler_params=pltpu.CompilerParams(needs_layout_passes=False),
        scratch_types=[
            pltpu.VMEM((XS_ROWS,), jnp.float32),
            pltpu.VMEM((CHUNK,), jnp.int32),
            pltpu.VMEM((CHUNK,), jnp.float32),
        ],
    )


@jax.jit
def kernel(expert_input, router_input, router_w, router_b,
           l1_w, l1_b, l2_w, l2_b, out_w, out_b):
    f32 = jnp.float32
    rb = router_b.astype(f32).reshape(1, E)
    idx, frac, avg, aux, z, ent, top, pref, cnts = _router(
        router_input.astype(f32), router_w.astype(f32), rb)

    xs, dst, te, tb = _dispatch()(idx.reshape(B), expert_input, pref, cnts)

    os = _grouped(te, tb, xs, l1_w, l1_b.reshape(E, 1, L2 + 1), l2_w,
                  l2_b.reshape(E, 1, L3), out_w, out_b.reshape(E, 1, 1))

    l3x = _unpermute()(os.reshape(XS_ROWS), dst).reshape(B, 1)

    zero = jnp.zeros((), dtype=expert_input.dtype)
    return (l3x, zero, aux.reshape(()), z.reshape(()), frac.reshape(E),
            avg.reshape(E), ent.reshape(()), top.reshape(()), zero)


# speedup vs baseline: 1.2461x; 1.0210x over previous
import functools
import math

import jax
import jax.numpy as jnp
from jax import lax
from jax.experimental import pallas as pl
from jax.experimental.pallas import tpu as pltpu
from jax.experimental.pallas import tpu_sc as plsc

E = 8
L2 = 512
L3 = 32
DR = 2048
DE = 2048
B = 4096

BT = 512
NBT = B // BT

TB = 512
TB_BITS = 9
T_MAX = B // TB + E - 1
TRASH_BLK = (B + E * TB) // TB
XS_ROWS = (TRASH_BLK + 1) * TB

NW = 32
CHUNK = B // NW
ROWB = 16
NVEC = B // 16


def _router_body(x_ref, rwt_ref, rb_ref,
                 idx_ref, frac_ref, avg_ref, aux_ref, z_ref, ent_ref, top_ref,
                 pref_ref, cnt_ref, run_ref):
    i = pl.program_id(0)
    x = x_ref[...]
    logits = lax.dot_general(x, rwt_ref[...], (((1,), (1,)), ((), ())),
                             preferred_element_type=jnp.float32)
    logits = logits + rb_ref[...]
    m = jnp.max(logits, axis=-1, keepdims=True)
    ex = jnp.exp(logits - m)
    s = jnp.sum(ex, axis=-1, keepdims=True)
    p = ex / s
    lse = m + jnp.log(s)

    iota = lax.broadcasted_iota(jnp.int32, (BT, E), 1)
    idx = jnp.min(jnp.where(logits == m, iota, E), axis=-1, keepdims=True)
    idx_ref[...] = idx

    iota16 = lax.broadcasted_iota(jnp.int32, (BT, 16), 1)
    oh16 = (iota16 == idx).astype(jnp.int32)
    subs = [jnp.sum(oh16[k * CHUNK:(k + 1) * CHUNK], axis=0, keepdims=True)
            for k in range(BT // CHUNK)]

    @pl.when(i == 0)
    def _init_run():
        run_ref[...] = jnp.zeros((1, 16), jnp.int32)

    rows = [run_ref[...]]
    for k in range(BT // CHUNK - 1):
        rows.append(rows[-1] + subs[k])
    pref_ref[0] = jnp.concatenate(rows, axis=0)
    run_ref[...] = rows[-1] + subs[BT // CHUNK - 1]

    @pl.when(i == NBT - 1)
    def _fin_cnt():
        cnt_ref[...] = run_ref[...]

    onehot = (iota == idx).astype(jnp.float32)
    part_p = jnp.sum(p, axis=0, keepdims=True)
    part_c = jnp.sum(onehot, axis=0, keepdims=True)
    part_z = jnp.sum(lse * lse)[None, None]
    part_e = jnp.sum(-p * jnp.log(jnp.maximum(p, 1e-9)))[None, None]
    part_t = jnp.sum(jnp.max(p, axis=-1))[None, None]

    @pl.when(i == 0)
    def _init():
        avg_ref[...] = part_p
        frac_ref[...] = part_c
        z_ref[...] = part_z
        ent_ref[...] = part_e
        top_ref[...] = part_t

    @pl.when(i != 0)
    def _acc():
        avg_ref[...] += part_p
        frac_ref[...] += part_c
        z_ref[...] += part_z
        ent_ref[...] += part_e
        top_ref[...] += part_t

    @pl.when(i == NBT - 1)
    def _fin():
        frac = frac_ref[...] / float(B)
        avg = avg_ref[...] / float(B)
        frac_ref[...] = frac
        avg_ref[...] = avg
        aux_ref[...] = (float(E) * jnp.sum(frac * avg))[None, None]
        z_ref[...] = z_ref[...] / float(B)
        ent_ref[...] = ent_ref[...] / (float(B) * math.log(float(E)))
        top_ref[...] = top_ref[...] / float(B)


def _router(router_input, rwt, rb):
    f32 = jnp.float32
    out_shapes = (
        jax.ShapeDtypeStruct((B, 1), jnp.int32),
        jax.ShapeDtypeStruct((1, E), f32),
        jax.ShapeDtypeStruct((1, E), f32),
        jax.ShapeDtypeStruct((1, 1), f32),
        jax.ShapeDtypeStruct((1, 1), f32),
        jax.ShapeDtypeStruct((1, 1), f32),
        jax.ShapeDtypeStruct((1, 1), f32),
        jax.ShapeDtypeStruct((NBT, BT // CHUNK, 16), jnp.int32),
        jax.ShapeDtypeStruct((1, 16), jnp.int32),
    )
    const = lambda i: (0, 0)
    return pl.pallas_call(
        _router_body,
        grid=(NBT,),
        in_specs=[
            pl.BlockSpec((BT, DR), lambda i: (i, 0)),
            pl.BlockSpec((E, DR), const),
            pl.BlockSpec((1, E), const),
        ],
        out_specs=(
            pl.BlockSpec((BT, 1), lambda i: (i, 0)),
            pl.BlockSpec((1, E), const),
            pl.BlockSpec((1, E), const),
            pl.BlockSpec((1, 1), const),
            pl.BlockSpec((1, 1), const),
            pl.BlockSpec((1, 1), const),
            pl.BlockSpec((1, 1), const),
            pl.BlockSpec((1, BT // CHUNK, 16), lambda i: (i, 0, 0)),
            pl.BlockSpec((1, 16), const),
        ),
        out_shape=out_shapes,
        scratch_shapes=[pltpu.VMEM((1, 16), jnp.int32)],
    )(router_input, rwt, rb)


def _dispatch_body(idx_hbm, x_hbm, pref_hbm, tot_hbm,
                   xs_hbm, dst_hbm, te_hbm, tb_hbm,
                   idx_all, cnt_ref, pfx_ref, tot_ref, dstc_ref, dstl_ref,
                   rows0_ref, rows1_ref, me_ref, mb_ref, sem_in, sem_out):
    c = lax.axis_index("c")
    s = lax.axis_index("s")
    w = s * 2 + c
    base_tok = w * CHUNK

    pltpu.sync_copy(idx_hbm.at[pl.ds(base_tok, CHUNK)], idx_all)
    pltpu.sync_copy(pref_hbm.at[w // 4, w % 4], pfx_ref)
    pltpu.sync_copy(tot_hbm.at[0], tot_ref)

    h_in0 = pltpu.make_async_copy(
        x_hbm.at[pl.ds(base_tok, ROWB)], rows0_ref, sem_in)
    h_in0.start()

    lanes = lax.iota(jnp.int32, 16)
    zero16 = jnp.zeros((16,), jnp.int32)
    total = tot_ref[...]
    prefix = pfx_ref[...]

    pc = ((total + (TB - 1)) >> TB_BITS) << TB_BITS
    seg_incl = plsc.cumsum(pc)
    seg_start = seg_incl - pc
    cnt_ref[...] = seg_start + prefix

    for j in range(CHUNK // 16):
        v = idx_all[pl.ds(j * 16, 16)]
        pos = plsc.load_gather(cnt_ref, [v])
        riv = zero16
        newcnt = cnt_ref[...]
        for e in range(E):
            mask = v == e
            cs = plsc.cumsum(mask.astype(jnp.int32))
            riv = jnp.where(mask, cs - 1, riv)
            newcnt = newcnt + jnp.where(lanes == e, cs[15], 0)
        cnt_ref[...] = newcnt
        dst_v = pos + riv
        dstc_ref[j] = dst_v
        dstl_ref[pl.ds(j * 16, 16)] = dst_v
    pltpu.sync_copy(dstl_ref, dst_hbm.at[pl.ds(base_tok, CHUNK)])

    bufs = (rows0_ref, rows1_ref)
    nb = CHUNK // ROWB
    h_out = [None] * nb
    h_in = [h_in0] + [None] * (nb - 1)
    for bi in range(nb):
        h_in[bi].wait()
        h_out[bi] = pltpu.make_async_copy(
            bufs[bi % 2], xs_hbm.at[dstc_ref.at[bi]], sem_out)
        h_out[bi].start()
        if bi + 1 < nb:
            if bi >= 1:
                h_out[bi - 1].wait()
            h_in[bi + 1] = pltpu.make_async_copy(
                x_hbm.at[pl.ds(base_tok + (bi + 1) * ROWB, ROWB)],
                bufs[(bi + 1) % 2], sem_in)
            h_in[bi + 1].start()
    h_out[nb - 2].wait()
    h_out[nb - 1].wait()

    @pl.when(w == 0)
    def _meta():
        nt = pc >> TB_BITS
        tiles_before = plsc.cumsum(nt) - nt
        segblk = seg_start >> TB_BITS
        for half in range(2):
            t_iota = lanes + half * 16
            me = jnp.zeros((16,), jnp.int32)
            mb = jnp.full((16,), TRASH_BLK, jnp.int32)
            for e in range(E):
                tb_e = jnp.sum(jnp.where(lanes == e, tiles_before, 0))
                nt_e = jnp.sum(jnp.where(lanes == e, nt, 0))
                sb_e = jnp.sum(jnp.where(lanes == e, segblk, 0))
                m = (t_iota >= tb_e) & (t_iota < tb_e + nt_e)
                me = jnp.where(m, e, me)
                mb = jnp.where(m, sb_e + t_iota - tb_e, mb)
            me_ref[pl.ds(half * 16, 16)] = me
            mb_ref[pl.ds(half * 16, 16)] = mb
        pltpu.sync_copy(me_ref, te_hbm)
        pltpu.sync_copy(mb_ref, tb_hbm)


@functools.cache
def _dispatch():
    return pl.kernel(
        _dispatch_body,
        out_type=(
            jax.ShapeDtypeStruct((XS_ROWS, DE), jnp.float32),
            jax.ShapeDtypeStruct((B,), jnp.int32),
            jax.ShapeDtypeStruct((NW,), jnp.int32),
            jax.ShapeDtypeStruct((NW,), jnp.int32),
        ),
        mesh=plsc.VectorSubcoreMesh(core_axis_name="c", subcore_axis_name="s"),
        compiler_params=pltpu.CompilerParams(
            needs_layout_passes=False, use_tc_tiling_on_sc=True),
        scratch_types=[
            pltpu.VMEM((CHUNK,), jnp.int32),
            pltpu.VMEM((16,), jnp.int32),
            pltpu.VMEM((16,), jnp.int32),
            pltpu.VMEM((16,), jnp.int32),
            pltpu.VMEM((CHUNK // ROWB, ROWB), jnp.int32),
            pltpu.VMEM((CHUNK,), jnp.int32),
            pltpu.VMEM((ROWB, DE), jnp.float32),
            pltpu.VMEM((ROWB, DE), jnp.float32),
            pltpu.VMEM((NW,), jnp.int32),
            pltpu.VMEM((NW,), jnp.int32),
            pltpu.SemaphoreType.DMA,
            pltpu.SemaphoreType.DMA,
        ],
    )


def _grouped_body(te_ref, tb_ref, x_ref, w1_ref, b1_ref, w2_ref, b2_ref,
                  ow_ref, ob_ref, out_ref):
    t = pl.program_id(0)

    @pl.when(tb_ref[t] != TRASH_BLK)
    def _work():
        nt = (((1,), (1,)), ((), ()))
        e_t = te_ref[t]
        x = x_ref[...]
        wa = w1_ref[:L2, e_t, :]
        wb = w1_ref[pl.ds(L2, 1), e_t, :]
        ha = lax.dot_general(x, wa, nt, preferred_element_type=jnp.float32)
        ha = ha + b1_ref[0][:, :L2]
        hx = (jnp.sum(x * wb, axis=1, keepdims=True)
              + b1_ref[0][:, L2:])
        pa = jnp.clip(ha * ha * (255.0 / 256.0), 0.0, 1.0)
        qa = jnp.clip(ha, 0.0, 1.0)
        w2 = w2_ref[0]
        y = (lax.dot_general(pa, w2[:, :L2], nt,
                             preferred_element_type=jnp.float32)
             + lax.dot_general(qa, w2[:, L2:], nt,
                               preferred_element_type=jnp.float32)
             + b2_ref[0])
        y = jnp.clip(y, 0.0, 1.0)
        o = jnp.sum(y * ow_ref[0], axis=1, keepdims=True)
        out_ref[...] = o + ob_ref[0] + hx


def _grouped(te, tb, xs, w1t, b1, w2t, b2, owt, ob):
    grid_spec = pltpu.PrefetchScalarGridSpec(
        num_scalar_prefetch=2,
        grid=(T_MAX,),
        in_specs=[
            pl.BlockSpec((TB, DE), lambda t, te, tb: (tb[t], 0)),
            pl.BlockSpec((L2 + 1, E, DE), lambda t, te, tb: (0, 0, 0)),
            pl.BlockSpec((1, 1, L2 + 1), lambda t, te, tb: (te[t], 0, 0)),
            pl.BlockSpec((1, L3, 2 * L2), lambda t, te, tb: (te[t], 0, 0)),
            pl.BlockSpec((1, 1, L3), lambda t, te, tb: (te[t], 0, 0)),
            pl.BlockSpec((1, 1, L3), lambda t, te, tb: (te[t], 0, 0)),
            pl.BlockSpec((1, 1, 1), lambda t, te, tb: (te[t], 0, 0)),
        ],
        out_specs=pl.BlockSpec((TB, 1), lambda t, te, tb: (tb[t], 0)),
    )
    return pl.pallas_call(
        _grouped_body,
        grid_spec=grid_spec,
        out_shape=jax.ShapeDtypeStruct((XS_ROWS, 1), jnp.float32),
    )(te, tb, xs, w1t, b1, w2t, b2, owt, ob)


def _unpermute_body(os_hbm, dst_hbm, out_hbm, os_v, dst_v, res_v):
    c = lax.axis_index("c")
    s = lax.axis_index("s")
    w = s * 2 + c
    base = w * CHUNK
    pltpu.sync_copy(os_hbm, os_v)
    pltpu.sync_copy(dst_hbm.at[pl.ds(base, CHUNK)], dst_v)
    for j in range(CHUNK // 16):
        iv = dst_v[pl.ds(j * 16, 16)]
        res_v[pl.ds(j * 16, 16)] = plsc.load_gather(os_v, [iv])
    pltpu.sync_copy(res_v, out_hbm.at[pl.ds(base, CHUNK)])


@functools.cache
def _unpermute():
    return pl.kernel(
        _unpermute_body,
        out_type=jax.ShapeDtypeStruct((B,), jnp.float32),
        mesh=plsc.VectorSubcoreMesh(core_axis_name="c", subcore_axis_name="s"),
        compiler_params=pltpu.CompilerParams(needs_layout_passes=False),
        scratch_types=[
            pltpu.VMEM((XS_ROWS,), jnp.float32),
            pltpu.VMEM((CHUNK,), jnp.int32),
            pltpu.VMEM((CHUNK,), jnp.float32),
        ],
    )


@jax.jit
def kernel(expert_input, router_input, router_w, router_b,
           l1_w, l1_b, l2_w, l2_b, out_w, out_b):
    f32 = jnp.float32
    rb = router_b.astype(f32).reshape(1, E)
    idx, frac, avg, aux, z, ent, top, pref, cnts = _router(
        router_input.astype(f32), router_w.astype(f32), rb)

    xs, dst, te, tb = _dispatch()(idx.reshape(B), expert_input, pref, cnts)

    os = _grouped(te, tb, xs, jnp.swapaxes(l1_w, 0, 1),
                  l1_b.reshape(E, 1, L2 + 1), l2_w,
                  l2_b.reshape(E, 1, L3), out_w, out_b.reshape(E, 1, 1))

    l3x = _unpermute()(os.reshape(XS_ROWS), dst).reshape(B, 1)

    zero = jnp.zeros((), dtype=expert_input.dtype)
    return (l3x, zero, aux.reshape(()), z.reshape(()), frac.reshape(E),
            avg.reshape(E), ent.reshape(()), top.reshape(()), zero)
